# Initial kernel scaffold; baseline (speedup 1.0000x reference)
import jax
import jax.numpy as jnp
from jax.experimental import pallas as pl


def kernel(feat, edge_index, edge_type, W_fc, W_self, attn_w):
    raise NotImplementedError("write your pallas kernel here")


# speedup vs baseline: 2.1618x; 2.1618x over previous
import functools

import jax
import jax.numpy as jnp
from jax import lax
from jax.experimental import pallas as pl
from jax.experimental.pallas import tpu as pltpu
from jax.experimental.pallas import tpu_sc as plsc

N = 10000
E = 160000
FIN = 256
FOUT = 128
H = 5
R = 20
HP = 16
NC = 2
NS = 16
NW = NC * NS
B = 128
EPAD = 163840
EW = EPAD // NW
NBATCH = EW // B
WT = 10
WR = N // WT
ZR = 200

_mesh = plsc.VectorSubcoreMesh(
    core_axis_name="c", subcore_axis_name="s", num_cores=NC, num_subcores=NS)


def _dense_body(feat_ref, wfc_ref, wself_ref, asrc_ref, adst_ref,
                z_ref, selfz_ref, ssrc_ref, sdst_ref):
    f = feat_ref[...]
    z = jnp.dot(f, wfc_ref[...], preferred_element_type=jnp.float32)
    z_ref[...] = z
    selfz_ref[...] = jnp.dot(f, wself_ref[...], preferred_element_type=jnp.float32)
    ssrc_ref[...] = jnp.dot(z, asrc_ref[...], preferred_element_type=jnp.float32)
    sdst_ref[...] = jnp.dot(z, adst_ref[...], preferred_element_type=jnp.float32)


def _dense(feat, wfc_t, wself_t, asrc, adst):
    bn = 1000
    grid = (N // bn,)
    return pl.pallas_call(
        _dense_body,
        grid=grid,
        in_specs=[
            pl.BlockSpec((bn, FIN), lambda i: (i, 0)),
            pl.BlockSpec((FIN, FOUT), lambda i: (0, 0)),
            pl.BlockSpec((FIN, H * FOUT), lambda i: (0, 0)),
            pl.BlockSpec((FOUT, R * HP), lambda i: (0, 0)),
            pl.BlockSpec((FOUT, R * HP), lambda i: (0, 0)),
        ],
        out_specs=[
            pl.BlockSpec((bn, FOUT), lambda i: (i, 0)),
            pl.BlockSpec((bn, H * FOUT), lambda i: (i, 0)),
            pl.BlockSpec((bn, R * HP), lambda i: (i, 0)),
            pl.BlockSpec((bn, R * HP), lambda i: (i, 0)),
        ],
        out_shape=[
            jax.ShapeDtypeStruct((N, FOUT), jnp.float32),
            jax.ShapeDtypeStruct((N, H * FOUT), jnp.float32),
            jax.ShapeDtypeStruct((N, R * HP), jnp.float32),
            jax.ShapeDtypeStruct((N, R * HP), jnp.float32),
        ],
    )(feat, wfc_t, wself_t, asrc, adst)


@functools.partial(
    pl.kernel,
    out_type=jax.ShapeDtypeStruct((EPAD * HP,), jnp.float32),
    mesh=_mesh,
    scratch_types=[
        pltpu.VMEM((B,), jnp.int32),
        pltpu.VMEM((B,), jnp.int32),
        pltpu.VMEM((B,), jnp.int32),
        pltpu.VMEM((B,), jnp.int32),
        pltpu.VMEM((B,), jnp.int32),
        pltpu.VMEM((B, 8 * HP), jnp.float32),
        pltpu.VMEM((B, 8 * HP), jnp.float32),
        pltpu.VMEM((B * HP,), jnp.float32),
        pltpu.SemaphoreType.DMA,
        pltpu.SemaphoreType.DMA,
    ],
)
def _attn(src_hbm, dst_hbm, et_hbm, ssrc_hbm, sdst_hbm, att_hbm,
          src_v, dst_v, et_v, gsrow, gdrow, srows, drows, attbuf,
          sem1, sem2):
    c = lax.axis_index("c")
    s = lax.axis_index("s")
    wid = c * NS + s

    def batch(j, carry):
        base = wid * EW + j * B
        pltpu.sync_copy(src_hbm.at[pl.ds(base, B)], src_v)
        pltpu.sync_copy(dst_hbm.at[pl.ds(base, B)], dst_v)
        pltpu.sync_copy(et_hbm.at[pl.ds(base, B)], et_v)

        def cidx(k, carry2):
            sl = pl.ds(k * 16, 16)
            e16 = et_v[sl]
            gs = src_v[sl] * R + e16
            gd = dst_v[sl] * R + e16
            gsrow[sl] = lax.shift_right_logical(gs, 3)
            gdrow[sl] = lax.shift_right_logical(gd, 3)
            src_v[sl] = lax.bitwise_and(gs, 7) * HP
            dst_v[sl] = lax.bitwise_and(gd, 7) * HP
            return carry2
        lax.fori_loop(0, B // 16, cidx, 0)

        cp1 = pltpu.async_copy(ssrc_hbm.at[gsrow], srows, sem1)
        cp2 = pltpu.async_copy(sdst_hbm.at[gdrow], drows, sem2)
        cp1.wait()
        cp2.wait()

        okf = jnp.where(base < E, jnp.float32(1.0), jnp.float32(0.0))
        okv = jnp.full((16,), okf, jnp.float32)

        def edge_att(k, carry2):
            sl16 = pl.ds(k * 16, 16)
            ss = src_v[sl16]
            dd = dst_v[sl16]
            for t in range(16):
                i = k * 16 + t
                v = srows[i, pl.ds(ss[t], 16)] + drows[i, pl.ds(dd[t], 16)]
                v = jnp.where(v > 0, v, v * jnp.float32(0.01))
                attbuf[pl.ds(i * HP, 16)] = v * okv
            return carry2
        lax.fori_loop(0, B // 16, edge_att, 0)

        pltpu.sync_copy(attbuf, att_hbm.at[pl.ds(base * HP, B * HP)])
        return carry
    lax.fori_loop(0, NBATCH, batch, 0)


@functools.partial(
    pl.kernel,
    out_type=jax.ShapeDtypeStruct((NC, H, N, FOUT), jnp.float32),
    mesh=_mesh,
    scratch_types=[
        pltpu.VMEM_SHARED((N, FOUT), jnp.float32),
        pltpu.VMEM((ZR, FOUT), jnp.float32),
        pltpu.VMEM((B, FOUT), jnp.float32),
        pltpu.VMEM((B,), jnp.int32),
        pltpu.VMEM((B,), jnp.int32),
        pltpu.VMEM((B * HP,), jnp.float32),
        pltpu.SemaphoreType.DMA,
    ],
)
def _agg(src_hbm, dst_hbm, att_hbm, z_hbm, out_hbm,
         agg, zbuf, zrows, src_v, dst_v, attrows, sem):
    c = lax.axis_index("c")
    s = lax.axis_index("s")
    wid = c * NS + s

    def zrow(i, carry):
        for f in range(FOUT // 16):
            zbuf[i, pl.ds(f * 16, 16)] = jnp.zeros((16,), jnp.float32)
        return carry
    lax.fori_loop(0, ZR, zrow, 0)

    for h in range(H):
        @pl.when(s < WT)
        def _():
            def zinit(k, carry):
                pltpu.sync_copy(zbuf, agg.at[pl.ds(s * WR + k * ZR, ZR)])
                return carry
            lax.fori_loop(0, WR // ZR, zinit, 0)
        plsc.subcore_barrier()

        def batch(j, carry):
            base = wid * EW + j * B
            pltpu.sync_copy(src_hbm.at[pl.ds(base, B)], src_v)
            pltpu.sync_copy(dst_hbm.at[pl.ds(base, B)], dst_v)
            pltpu.sync_copy(att_hbm.at[pl.ds(base * HP, B * HP)], attrows)
            pltpu.async_copy(z_hbm.at[src_v], zrows, sem).wait()

            def scale16(k, carry2):
                ebase = k * 16
                for t in range(16):
                    arow = attrows[pl.ds((ebase + t) * HP, 16)]
                    av = jnp.full((16,), arow[h], jnp.float32)
                    for f in range(FOUT // 16):
                        sl = pl.ds(f * 16, 16)
                        zrows[ebase + t, sl] = zrows[ebase + t, sl] * av
                return carry2
            lax.fori_loop(0, B // 16, scale16, 0)

            pltpu.sync_copy(zrows, agg.at[dst_v], add=True)
            return carry
        lax.fori_loop(0, NBATCH, batch, 0)

        plsc.subcore_barrier()

        @pl.when(s < WT)
        def _():
            pltpu.sync_copy(agg.at[pl.ds(s * WR, WR)],
                            out_hbm.at[c, h, pl.ds(s * WR, WR)])
        plsc.subcore_barrier()


def kernel(feat, edge_index, edge_type, W_fc, W_self, attn_w):
    src = edge_index[0]
    dst = edge_index[1]
    pad = EPAD - E
    srcp = jnp.pad(src, (0, pad))
    dstp = jnp.pad(dst, (0, pad))
    etp = jnp.pad(edge_type, (0, pad))

    wfc_t = W_fc.T
    wself_t = W_self.T
    ap = jnp.pad(attn_w, ((0, 0), (0, 0), (0, HP - H)))
    asrc = ap[:, :FOUT, :].transpose(1, 0, 2).reshape(FOUT, R * HP)
    adst = ap[:, FOUT:, :].transpose(1, 0, 2).reshape(FOUT, R * HP)

    z, selfz, ssrc, sdst = _dense(feat, wfc_t, wself_t, asrc, adst)
    ssrc_t = ssrc.reshape(N * R // 8, 8 * HP)
    sdst_t = sdst.reshape(N * R // 8, 8 * HP)

    att = _attn(srcp, dstp, etp, ssrc_t, sdst_t)
    part = _agg(srcp, dstp, att, z)

    aggsum = part[0] + part[1]
    return aggsum.transpose(1, 0, 2).reshape(N, H * FOUT) + selfz


# speedup vs baseline: 3.0380x; 1.4053x over previous
import functools

import jax
import jax.numpy as jnp
from jax import lax
from jax.experimental import pallas as pl
from jax.experimental.pallas import tpu as pltpu
from jax.experimental.pallas import tpu_sc as plsc

N = 10000
E = 160000
FIN = 256
FOUT = 128
H = 5
R = 20
HP = 16
NC = 2
NS = 16
NW = NC * NS
B = 128
EPAD = 163840
EW = EPAD // NW
NBATCH = EW // B
NPAIR = NBATCH // 2
WT = 10
WR = N // WT
ZR = 200
CB = 8
CE = CB * B

_mesh = plsc.VectorSubcoreMesh(
    core_axis_name="c", subcore_axis_name="s", num_cores=NC, num_subcores=NS)


def _dense_body(feat_ref, wfc_ref, wself_ref, asrc_ref, adst_ref,
                z_ref, selfz_ref, ssrc_ref, sdst_ref):
    f = feat_ref[...]
    z = jnp.dot(f, wfc_ref[...], preferred_element_type=jnp.float32)
    z_ref[...] = z
    selfz_ref[...] = jnp.dot(f, wself_ref[...], preferred_element_type=jnp.float32)
    ssrc_ref[...] = jnp.dot(z, asrc_ref[...], preferred_element_type=jnp.float32)
    sdst_ref[...] = jnp.dot(z, adst_ref[...], preferred_element_type=jnp.float32)


def _dense(feat, wfc_t, wself_t, asrc, adst):
    bn = 1000
    grid = (N // bn,)
    return pl.pallas_call(
        _dense_body,
        grid=grid,
        in_specs=[
            pl.BlockSpec((bn, FIN), lambda i: (i, 0)),
            pl.BlockSpec((FIN, FOUT), lambda i: (0, 0)),
            pl.BlockSpec((FIN, H * FOUT), lambda i: (0, 0)),
            pl.BlockSpec((FOUT, R * HP), lambda i: (0, 0)),
            pl.BlockSpec((FOUT, R * HP), lambda i: (0, 0)),
        ],
        out_specs=[
            pl.BlockSpec((bn, FOUT), lambda i: (i, 0)),
            pl.BlockSpec((bn, H * FOUT), lambda i: (i, 0)),
            pl.BlockSpec((bn, R * HP), lambda i: (i, 0)),
            pl.BlockSpec((bn, R * HP), lambda i: (i, 0)),
        ],
        out_shape=[
            jax.ShapeDtypeStruct((N, FOUT), jnp.float32),
            jax.ShapeDtypeStruct((N, H * FOUT), jnp.float32),
            jax.ShapeDtypeStruct((N, R * HP), jnp.float32),
            jax.ShapeDtypeStruct((N, R * HP), jnp.float32),
        ],
    )(feat, wfc_t, wself_t, asrc, adst)


@functools.partial(
    pl.kernel,
    out_type=jax.ShapeDtypeStruct((EPAD * HP,), jnp.float32),
    mesh=_mesh,
    scratch_types=[
        pltpu.VMEM((NBATCH, B), jnp.int32),
        pltpu.VMEM((NBATCH, B), jnp.int32),
        pltpu.VMEM((NBATCH, B), jnp.int32),
        pltpu.VMEM((NBATCH, B), jnp.int32),
        pltpu.VMEM((B, 8 * HP), jnp.float32),
        pltpu.VMEM((B, 8 * HP), jnp.float32),
        pltpu.VMEM((B, 8 * HP), jnp.float32),
        pltpu.VMEM((B, 8 * HP), jnp.float32),
        pltpu.VMEM((B * HP,), jnp.float32),
        pltpu.VMEM((B * HP,), jnp.float32),
        pltpu.SemaphoreType.DMA,
        pltpu.SemaphoreType.DMA,
        pltpu.SemaphoreType.DMA,
        pltpu.SemaphoreType.DMA,
        pltpu.SemaphoreType.DMA,
        pltpu.SemaphoreType.DMA,
    ],
)
def _attn(src_hbm, dst_hbm, et_hbm, ssrc_hbm, sdst_hbm, att_hbm,
          gsrow, gdrow, sslot, dslot, srows0, srows1, drows0, drows1,
          attbuf0, attbuf1, sgsem0, sgsem1, dgsem0, dgsem1, asem0, asem1):
    c = lax.axis_index("c")
    s = lax.axis_index("s")
    wid = c * NS + s
    row0 = wid * NBATCH

    pltpu.sync_copy(src_hbm.at[pl.ds(row0, NBATCH), :], gsrow)
    pltpu.sync_copy(dst_hbm.at[pl.ds(row0, NBATCH), :], gdrow)
    pltpu.sync_copy(et_hbm.at[pl.ds(row0, NBATCH), :], sslot)

    def pidx(m, carry):
        def inner(k, carry2):
            sl = pl.ds(k * 16, 16)
            e16 = sslot[m, sl]
            gs = gsrow[m, sl] * R + e16
            gd = gdrow[m, sl] * R + e16
            gsrow[m, sl] = lax.shift_right_logical(gs, 3)
            gdrow[m, sl] = lax.shift_right_logical(gd, 3)
            sslot[m, sl] = lax.bitwise_and(gs, 7) * HP
            dslot[m, sl] = lax.bitwise_and(gd, 7) * HP
            return carry2
        return lax.fori_loop(0, B // 16, inner, carry)
    lax.fori_loop(0, NBATCH, pidx, 0)

    def sgather(j, buf, sem):
        return pltpu.make_async_copy(ssrc_hbm.at[gsrow.at[j]], buf, sem)

    def dgather(j, buf, sem):
        return pltpu.make_async_copy(sdst_hbm.at[gdrow.at[j]], buf, sem)

    def awrite(j, buf, sem):
        base = (row0 + j) * (B * HP)
        return pltpu.make_async_copy(buf, att_hbm.at[pl.ds(base, B * HP)], sem)

    def edge_att(j, sbuf, dbuf, abuf):
        okf = jnp.where(row0 + j < E // B, jnp.float32(1.0), jnp.float32(0.0))
        okv = jnp.full((16,), okf, jnp.float32)

        def ea(k, carry):
            sl16 = pl.ds(k * 16, 16)
            ss = sslot[j, sl16]
            dd = dslot[j, sl16]
            for t in range(16):
                i = k * 16 + t
                v = sbuf[i, pl.ds(ss[t], 16)] + dbuf[i, pl.ds(dd[t], 16)]
                v = jnp.where(v > 0, v, v * jnp.float32(0.01))
                abuf[pl.ds(i * HP, 16)] = v * okv
            return carry
        lax.fori_loop(0, B // 16, ea, 0)

    sgather(0, srows0, sgsem0).start()
    dgather(0, drows0, dgsem0).start()

    def pair(jj, carry):
        j0 = 2 * jj
        j1 = j0 + 1
        sgather(j1, srows1, sgsem1).start()
        dgather(j1, drows1, dgsem1).start()
        sgather(j0, srows0, sgsem0).wait()
        dgather(j0, drows0, dgsem0).wait()

        @pl.when(jj > 0)
        def _():
            awrite(j0, attbuf0, asem0).wait()
        edge_att(j0, srows0, drows0, attbuf0)
        awrite(j0, attbuf0, asem0).start()

        @pl.when(jj < NPAIR - 1)
        def _():
            sgather(j0 + 2, srows0, sgsem0).start()
            dgather(j0 + 2, drows0, dgsem0).start()
        sgather(j1, srows1, sgsem1).wait()
        dgather(j1, drows1, dgsem1).wait()

        @pl.when(jj > 0)
        def _():
            awrite(j1, attbuf1, asem1).wait()
        edge_att(j1, srows1, drows1, attbuf1)
        awrite(j1, attbuf1, asem1).start()
        return carry
    lax.fori_loop(0, NPAIR, pair, 0)

    awrite(NBATCH - 2, attbuf0, asem0).wait()
    awrite(NBATCH - 1, attbuf1, asem1).wait()


@functools.partial(
    pl.kernel,
    out_type=jax.ShapeDtypeStruct((NC, H, N, FOUT), jnp.float32),
    mesh=_mesh,
    scratch_types=[
        pltpu.VMEM_SHARED((N, FOUT), jnp.float32),
        pltpu.VMEM((NBATCH, B), jnp.int32),
        pltpu.VMEM((NBATCH, B), jnp.int32),
        pltpu.VMEM((B, FOUT), jnp.float32),
        pltpu.VMEM((B, FOUT), jnp.float32),
        pltpu.VMEM((B * HP,), jnp.float32),
        pltpu.VMEM((B * HP,), jnp.float32),
        pltpu.SemaphoreType.DMA,
        pltpu.SemaphoreType.DMA,
        pltpu.SemaphoreType.DMA,
        pltpu.SemaphoreType.DMA,
        pltpu.SemaphoreType.DMA,
        pltpu.SemaphoreType.DMA,
    ],
)
def _agg(src_hbm, dst_hbm, att_hbm, z_hbm, out_hbm,
         agg, src2, dst2, zrows0, zrows1, attb0, attb1,
         gsem0, gsem1, ssem0, ssem1, atsem0, atsem1):
    c = lax.axis_index("c")
    s = lax.axis_index("s")
    wid = c * NS + s
    row0 = wid * NBATCH

    pltpu.sync_copy(src_hbm.at[pl.ds(row0, NBATCH), :], src2)
    pltpu.sync_copy(dst_hbm.at[pl.ds(row0, NBATCH), :], dst2)

    def zgather(j, buf, sem):
        return pltpu.make_async_copy(z_hbm.at[src2.at[j]], buf, sem)

    def aload(j, buf, sem):
        base = (row0 + j) * (B * HP)
        return pltpu.make_async_copy(att_hbm.at[pl.ds(base, B * HP)], buf, sem)

    def zscatter_start(j, buf, sem):
        pltpu.async_copy(buf, agg.at[dst2.at[j]], sem, add=True)

    def zscatter_wait(j, buf, sem):
        pltpu.make_async_copy(buf, agg.at[dst2.at[j]], sem).wait()

    for h in range(H):
        def zclr(i, carry):
            for f in range(FOUT // 16):
                zrows0[i, pl.ds(f * 16, 16)] = jnp.zeros((16,), jnp.float32)
            return carry
        lax.fori_loop(0, B, zclr, 0)

        @pl.when(s < WT)
        def _():
            def zinit(k, carry):
                pltpu.sync_copy(zrows0.at[pl.ds(0, 100)],
                                agg.at[pl.ds(s * WR + k * 100, 100)])
                return carry
            lax.fori_loop(0, WR // 100, zinit, 0)
        plsc.subcore_barrier()

        def scale(zref, aref):
            def s16(k, carry):
                ebase = k * 16
                for t in range(16):
                    arow = aref[pl.ds((ebase + t) * HP, 16)]
                    av = jnp.full((16,), arow[h], jnp.float32)
                    for f in range(FOUT // 16):
                        sl = pl.ds(f * 16, 16)
                        zref[ebase + t, sl] = zref[ebase + t, sl] * av
                return carry
            lax.fori_loop(0, B // 16, s16, 0)

        zgather(0, zrows0, gsem0).start()
        aload(0, attb0, atsem0).start()

        def pair(jj, carry):
            j0 = 2 * jj
            j1 = j0 + 1

            @pl.when(jj > 0)
            def _():
                zscatter_wait(j0 - 1, zrows1, ssem1)
            zgather(j1, zrows1, gsem1).start()
            aload(j1, attb1, atsem1).start()
            zgather(j0, zrows0, gsem0).wait()
            aload(j0, attb0, atsem0).wait()
            scale(zrows0, attb0)
            zscatter_start(j0, zrows0, ssem0)

            zgather(j1, zrows1, gsem1).wait()
            aload(j1, attb1, atsem1).wait()
            scale(zrows1, attb1)
            zscatter_wait(j0, zrows0, ssem0)

            @pl.when(jj < NPAIR - 1)
            def _():
                zgather(j0 + 2, zrows0, gsem0).start()
                aload(j0 + 2, attb0, atsem0).start()
            zscatter_start(j1, zrows1, ssem1)
            return carry
        lax.fori_loop(0, NPAIR, pair, 0)

        zscatter_wait(NBATCH - 1, zrows1, ssem1)
        plsc.subcore_barrier()

        @pl.when(s < WT)
        def _():
            pltpu.sync_copy(agg.at[pl.ds(s * WR, WR)],
                            out_hbm.at[c, h, pl.ds(s * WR, WR)])
        plsc.subcore_barrier()


def kernel(feat, edge_index, edge_type, W_fc, W_self, attn_w):
    src = edge_index[0]
    dst = edge_index[1]
    pad = EPAD - E
    src2 = jnp.pad(src, (0, pad)).reshape(EPAD // B, B)
    dst2 = jnp.pad(dst, (0, pad)).reshape(EPAD // B, B)
    et2 = jnp.pad(edge_type, (0, pad)).reshape(EPAD // B, B)

    wfc_t = W_fc.T
    wself_t = W_self.T
    ap = jnp.pad(attn_w, ((0, 0), (0, 0), (0, HP - H)))
    asrc = ap[:, :FOUT, :].transpose(1, 0, 2).reshape(FOUT, R * HP)
    adst = ap[:, FOUT:, :].transpose(1, 0, 2).reshape(FOUT, R * HP)

    z, selfz, ssrc, sdst = _dense(feat, wfc_t, wself_t, asrc, adst)
    ssrc_t = ssrc.reshape(N * R // 8, 8 * HP)
    sdst_t = sdst.reshape(N * R // 8, 8 * HP)

    att = _attn(src2, dst2, et2, ssrc_t, sdst_t)
    part = _agg(src2, dst2, att, z)

    aggsum = part[0] + part[1]
    return aggsum.transpose(1, 0, 2).reshape(N, H * FOUT) + selfz


# speedup vs baseline: 3.2382x; 1.0659x over previous
import functools

import jax
import jax.numpy as jnp
from jax import lax
from jax.experimental import pallas as pl
from jax.experimental.pallas import tpu as pltpu
from jax.experimental.pallas import tpu_sc as plsc

N = 10000
E = 160000
FIN = 256
FOUT = 128
H = 5
R = 20
HP = 16
NC = 2
NS = 16
NW = NC * NS
B = 128
EPAD = 163840
EW = EPAD // NW
NBATCH = EW // B
NPAIR = NBATCH // 2
WT = 10
WR = N // WT
ZR = 200
CB = 8
CE = CB * B

_mesh = plsc.VectorSubcoreMesh(
    core_axis_name="c", subcore_axis_name="s", num_cores=NC, num_subcores=NS)


def _dense_body(feat_ref, wfc_ref, wself_ref, asrc_ref, adst_ref,
                z_ref, selfz_ref, ssrc_ref, sdst_ref):
    f = feat_ref[...]
    z = jnp.dot(f, wfc_ref[...], preferred_element_type=jnp.float32)
    z_ref[...] = z
    selfz_ref[...] = jnp.dot(f, wself_ref[...], preferred_element_type=jnp.float32)
    ssrc_ref[...] = jnp.dot(z, asrc_ref[...], preferred_element_type=jnp.float32)
    sdst_ref[...] = jnp.dot(z, adst_ref[...], preferred_element_type=jnp.float32)


def _dense(feat, wfc_t, wself_t, asrc, adst):
    bn = 1000
    grid = (N // bn,)
    return pl.pallas_call(
        _dense_body,
        grid=grid,
        in_specs=[
            pl.BlockSpec((bn, FIN), lambda i: (i, 0)),
            pl.BlockSpec((FIN, FOUT), lambda i: (0, 0)),
            pl.BlockSpec((FIN, H * FOUT), lambda i: (0, 0)),
            pl.BlockSpec((FOUT, R * HP), lambda i: (0, 0)),
            pl.BlockSpec((FOUT, R * HP), lambda i: (0, 0)),
        ],
        out_specs=[
            pl.BlockSpec((bn, FOUT), lambda i: (i, 0)),
            pl.BlockSpec((bn, H * FOUT), lambda i: (i, 0)),
            pl.BlockSpec((bn, R * HP), lambda i: (i, 0)),
            pl.BlockSpec((bn, R * HP), lambda i: (i, 0)),
        ],
        out_shape=[
            jax.ShapeDtypeStruct((N, FOUT), jnp.float32),
            jax.ShapeDtypeStruct((N, H * FOUT), jnp.float32),
            jax.ShapeDtypeStruct((N, R * HP), jnp.float32),
            jax.ShapeDtypeStruct((N, R * HP), jnp.float32),
        ],
    )(feat, wfc_t, wself_t, asrc, adst)


@functools.partial(
    pl.kernel,
    out_type=jax.ShapeDtypeStruct((EPAD * HP,), jnp.float32),
    mesh=_mesh,
    scratch_types=[
        pltpu.VMEM((NBATCH, B), jnp.int32),
        pltpu.VMEM((NBATCH, B), jnp.int32),
        pltpu.VMEM((NBATCH, B), jnp.int32),
        pltpu.VMEM((NBATCH, B), jnp.int32),
        pltpu.VMEM((B, 8 * HP), jnp.float32),
        pltpu.VMEM((B, 8 * HP), jnp.float32),
        pltpu.VMEM((B, 8 * HP), jnp.float32),
        pltpu.VMEM((B, 8 * HP), jnp.float32),
        pltpu.VMEM((B * HP,), jnp.float32),
        pltpu.VMEM((B * HP,), jnp.float32),
        pltpu.SemaphoreType.DMA,
        pltpu.SemaphoreType.DMA,
        pltpu.SemaphoreType.DMA,
        pltpu.SemaphoreType.DMA,
        pltpu.SemaphoreType.DMA,
        pltpu.SemaphoreType.DMA,
    ],
)
def _attn(src_hbm, dst_hbm, et_hbm, ssrc_hbm, sdst_hbm, att_hbm,
          gsrow, gdrow, sslot, dslot, srows0, srows1, drows0, drows1,
          attbuf0, attbuf1, sgsem0, sgsem1, dgsem0, dgsem1, asem0, asem1):
    c = lax.axis_index("c")
    s = lax.axis_index("s")
    wid = c * NS + s
    row0 = wid * NBATCH

    pltpu.sync_copy(src_hbm.at[pl.ds(row0, NBATCH), :], gsrow)
    pltpu.sync_copy(dst_hbm.at[pl.ds(row0, NBATCH), :], gdrow)
    pltpu.sync_copy(et_hbm.at[pl.ds(row0, NBATCH), :], sslot)

    def pidx(m, carry):
        def inner(k, carry2):
            sl = pl.ds(k * 16, 16)
            e16 = sslot[m, sl]
            gs = gsrow[m, sl] * R + e16
            gd = gdrow[m, sl] * R + e16
            gsrow[m, sl] = lax.shift_right_logical(gs, 3)
            gdrow[m, sl] = lax.shift_right_logical(gd, 3)
            sslot[m, sl] = lax.bitwise_and(gs, 7) * HP
            dslot[m, sl] = lax.bitwise_and(gd, 7) * HP
            return carry2
        return lax.fori_loop(0, B // 16, inner, carry)
    lax.fori_loop(0, NBATCH, pidx, 0)

    def sgather(j, buf, sem):
        return pltpu.make_async_copy(ssrc_hbm.at[gsrow.at[j]], buf, sem)

    def dgather(j, buf, sem):
        return pltpu.make_async_copy(sdst_hbm.at[gdrow.at[j]], buf, sem)

    def awrite(j, buf, sem):
        base = (row0 + j) * (B * HP)
        return pltpu.make_async_copy(buf, att_hbm.at[pl.ds(base, B * HP)], sem)

    def edge_att(j, sbuf, dbuf, abuf):
        okf = jnp.where(row0 + j < E // B, jnp.float32(1.0), jnp.float32(0.0))
        okv = jnp.full((16,), okf, jnp.float32)

        def ea(k, carry):
            sl16 = pl.ds(k * 16, 16)
            ss = sslot[j, sl16]
            dd = dslot[j, sl16]
            for t in range(16):
                i = k * 16 + t
                v = sbuf[i, pl.ds(ss[t], 16)] + dbuf[i, pl.ds(dd[t], 16)]
                v = jnp.where(v > 0, v, v * jnp.float32(0.01))
                abuf[pl.ds(i * HP, 16)] = v * okv
            return carry
        lax.fori_loop(0, B // 16, ea, 0)

    sgather(0, srows0, sgsem0).start()
    dgather(0, drows0, dgsem0).start()

    def pair(jj, carry):
        j0 = 2 * jj
        j1 = j0 + 1
        sgather(j1, srows1, sgsem1).start()
        dgather(j1, drows1, dgsem1).start()
        sgather(j0, srows0, sgsem0).wait()
        dgather(j0, drows0, dgsem0).wait()

        @pl.when(jj > 0)
        def _():
            awrite(j0, attbuf0, asem0).wait()
        edge_att(j0, srows0, drows0, attbuf0)
        awrite(j0, attbuf0, asem0).start()

        @pl.when(jj < NPAIR - 1)
        def _():
            sgather(j0 + 2, srows0, sgsem0).start()
            dgather(j0 + 2, drows0, dgsem0).start()
        sgather(j1, srows1, sgsem1).wait()
        dgather(j1, drows1, dgsem1).wait()

        @pl.when(jj > 0)
        def _():
            awrite(j1, attbuf1, asem1).wait()
        edge_att(j1, srows1, drows1, attbuf1)
        awrite(j1, attbuf1, asem1).start()
        return carry
    lax.fori_loop(0, NPAIR, pair, 0)

    awrite(NBATCH - 2, attbuf0, asem0).wait()
    awrite(NBATCH - 1, attbuf1, asem1).wait()


@functools.partial(
    pl.kernel,
    out_type=jax.ShapeDtypeStruct((NC, N, H * FOUT), jnp.float32),
    mesh=_mesh,
    scratch_types=[
        pltpu.VMEM_SHARED((N, FOUT), jnp.float32),
        pltpu.VMEM((NBATCH, B), jnp.int32),
        pltpu.VMEM((NBATCH, B), jnp.int32),
        pltpu.VMEM((B, FOUT), jnp.float32),
        pltpu.VMEM((B, FOUT), jnp.float32),
        pltpu.VMEM((B * HP,), jnp.float32),
        pltpu.VMEM((B * HP,), jnp.float32),
        pltpu.SemaphoreType.DMA,
        pltpu.SemaphoreType.DMA,
        pltpu.SemaphoreType.DMA,
        pltpu.SemaphoreType.DMA,
        pltpu.SemaphoreType.DMA,
        pltpu.SemaphoreType.DMA,
    ],
)
def _agg(src_hbm, dst_hbm, att_hbm, z_hbm, selfz_hbm, out_hbm,
         agg, src2, dst2, zrows0, zrows1, attb0, attb1,
         gsem0, gsem1, ssem0, ssem1, atsem0, atsem1):
    c = lax.axis_index("c")
    s = lax.axis_index("s")
    wid = c * NS + s
    row0 = wid * NBATCH
    rbase = s * 624

    def zclr(i, carry):
        for f in range(FOUT // 16):
            zrows0[i, pl.ds(f * 16, 16)] = jnp.zeros((16,), jnp.float32)
        return carry

    def winit(h):
        @pl.when(c == 0)
        def _():
            pltpu.sync_copy(
                selfz_hbm.at[pl.ds(rbase, 624), pl.ds(h * FOUT, FOUT)],
                agg.at[pl.ds(rbase, 624)])

            @pl.when(s == NS - 1)
            def _():
                pltpu.sync_copy(
                    selfz_hbm.at[pl.ds(9984, 16), pl.ds(h * FOUT, FOUT)],
                    agg.at[pl.ds(9984, 16)])

        @pl.when(c == 1)
        def _():
            for k in range(4):
                pltpu.sync_copy(zrows0.at[pl.ds(0, 128)],
                                agg.at[pl.ds(rbase + k * 128, 128)])
            pltpu.sync_copy(zrows0.at[pl.ds(0, 112)],
                            agg.at[pl.ds(rbase + 512, 112)])

            @pl.when(s == NS - 1)
            def _():
                pltpu.sync_copy(zrows0.at[pl.ds(0, 16)],
                                agg.at[pl.ds(9984, 16)])

    def wback(h):
        pltpu.sync_copy(agg.at[pl.ds(rbase, 624)],
                        out_hbm.at[c, pl.ds(rbase, 624), pl.ds(h * FOUT, FOUT)])

        @pl.when(s == NS - 1)
        def _():
            pltpu.sync_copy(agg.at[pl.ds(9984, 16)],
                            out_hbm.at[c, pl.ds(9984, 16), pl.ds(h * FOUT, FOUT)])

    pltpu.sync_copy(src_hbm.at[pl.ds(row0, NBATCH), :], src2)
    pltpu.sync_copy(dst_hbm.at[pl.ds(row0, NBATCH), :], dst2)

    def zgather(j, buf, sem):
        return pltpu.make_async_copy(z_hbm.at[src2.at[j]], buf, sem)

    def aload(j, buf, sem):
        base = (row0 + j) * (B * HP)
        return pltpu.make_async_copy(att_hbm.at[pl.ds(base, B * HP)], buf, sem)

    def zscatter_start(j, buf, sem):
        pltpu.async_copy(buf, agg.at[dst2.at[j]], sem, add=True)

    def zscatter_wait(j, buf, sem):
        pltpu.make_async_copy(buf, agg.at[dst2.at[j]], sem).wait()

    lax.fori_loop(0, B, zclr, 0)
    winit(0)
    plsc.subcore_barrier()

    for h in range(H):

        def scale(zref, aref):
            def s16(k, carry):
                ebase = k * 16
                for t in range(16):
                    arow = aref[pl.ds((ebase + t) * HP, 16)]
                    av = jnp.full((16,), arow[h], jnp.float32)
                    for f in range(FOUT // 16):
                        sl = pl.ds(f * 16, 16)
                        zref[ebase + t, sl] = zref[ebase + t, sl] * av
                return carry
            lax.fori_loop(0, B // 16, s16, 0)

        zgather(0, zrows0, gsem0).start()
        aload(0, attb0, atsem0).start()

        def pair(jj, carry):
            j0 = 2 * jj
            j1 = j0 + 1

            @pl.when(jj > 0)
            def _():
                zscatter_wait(j0 - 1, zrows1, ssem1)
            zgather(j1, zrows1, gsem1).start()
            aload(j1, attb1, atsem1).start()
            zgather(j0, zrows0, gsem0).wait()
            aload(j0, attb0, atsem0).wait()
            scale(zrows0, attb0)
            zscatter_start(j0, zrows0, ssem0)

            zgather(j1, zrows1, gsem1).wait()
            aload(j1, attb1, atsem1).wait()
            scale(zrows1, attb1)
            zscatter_wait(j0, zrows0, ssem0)

            @pl.when(jj < NPAIR - 1)
            def _():
                zgather(j0 + 2, zrows0, gsem0).start()
                aload(j0 + 2, attb0, atsem0).start()
            zscatter_start(j1, zrows1, ssem1)
            return carry
        lax.fori_loop(0, NPAIR, pair, 0)

        zscatter_wait(NBATCH - 1, zrows1, ssem1)
        plsc.subcore_barrier()
        wback(h)
        if h < H - 1:
            lax.fori_loop(0, B, zclr, 0)
            winit(h + 1)
        plsc.subcore_barrier()


def kernel(feat, edge_index, edge_type, W_fc, W_self, attn_w):
    src = edge_index[0]
    dst = edge_index[1]
    pad = EPAD - E
    src2 = jnp.pad(src, (0, pad)).reshape(EPAD // B, B)
    dst2 = jnp.pad(dst, (0, pad)).reshape(EPAD // B, B)
    et2 = jnp.pad(edge_type, (0, pad)).reshape(EPAD // B, B)

    wfc_t = W_fc.T
    wself_t = W_self.T
    ap = jnp.pad(attn_w, ((0, 0), (0, 0), (0, HP - H)))
    asrc = ap[:, :FOUT, :].transpose(1, 0, 2).reshape(FOUT, R * HP)
    adst = ap[:, FOUT:, :].transpose(1, 0, 2).reshape(FOUT, R * HP)

    z, selfz, ssrc, sdst = _dense(feat, wfc_t, wself_t, asrc, adst)
    ssrc_t = ssrc.reshape(N * R // 8, 8 * HP)
    sdst_t = sdst.reshape(N * R // 8, 8 * HP)

    att = _attn(src2, dst2, et2, ssrc_t, sdst_t)
    part = _agg(src2, dst2, att, z, selfz)

    return part[0] + part[1]
